# Initial kernel scaffold; baseline (speedup 1.0000x reference)
import jax
import jax.numpy as jnp
from jax.experimental import pallas as pl


def kernel(x, edge_index, Wm, Wu, bu, gamma, beta):
    raise NotImplementedError("write your pallas kernel here")


# speedup vs baseline: 1.4960x; 1.4960x over previous
import functools
import math

import jax
import jax.numpy as jnp
from jax import lax
from jax.experimental import pallas as pl
from jax.experimental.pallas import tpu as pltpu
from jax.experimental.pallas import tpu_sc as plsc

N = 100000
D = 128
E = 600000
NS = 16
NC = 2
NW = NS * NC

EP = 614400
TEDGE = EP // NW
ECH = 960
NECH = TEDGE // ECH
NVEC = ECH // 16

BK = 3072
NB = 33
NBR = NB + 1
TRASH = BK
SEG = 112
NBLK = 9
CAPB = NBLK * SEG

ACC = 3200
ART = ACC // NS
NPASS = 17


def _sc_partition(src1, dst1, zcap, tcap):
    mesh = plsc.VectorSubcoreMesh(core_axis_name="c", subcore_axis_name="s")

    @functools.partial(
        pl.kernel,
        out_type=[
            jax.ShapeDtypeStruct((NW, NB, 1, CAPB), jnp.int32),
            jax.ShapeDtypeStruct((NW, NB, 1, CAPB), jnp.int32),
            jax.ShapeDtypeStruct((NW, 1, 64), jnp.int32),
        ],
        mesh=mesh,
        compiler_params=pltpu.CompilerParams(needs_layout_passes=False),
        scratch_types=[
            pltpu.VMEM((ECH,), jnp.int32),
            pltpu.VMEM((ECH,), jnp.int32),
            pltpu.VMEM((NBR * CAPB,), jnp.int32),
            pltpu.VMEM((NBR * CAPB,), jnp.int32),
            pltpu.VMEM((64,), jnp.int32),
        ],
    )
    def k(src_h, dst_h, z_h, t_h, sl_h, dl_h, cn_h,
          ssrc, sdst, sflat, dflat, cntv):
        cid = lax.axis_index("c")
        sid = lax.axis_index("s")
        wid = sid * NC + cid
        eoff = wid * TEDGE

        for b in range(NBR):
            pltpu.sync_copy(z_h, sflat.at[pl.ds(b * CAPB, CAPB)])
            pltpu.sync_copy(t_h, dflat.at[pl.ds(b * CAPB, CAPB)])
        zeros16 = jnp.zeros((16,), jnp.int32)
        for i in range(4):
            cntv[pl.ds(i * 16, 16)] = zeros16
        ones16 = jnp.ones((16,), jnp.int32)

        r0 = jnp.min(plsc.scan_count(zeros16)[0])

        def chunk_body(ch, _):
            pltpu.sync_copy(src_h.at[pl.ds(eoff + ch * ECH, ECH)], ssrc)
            pltpu.sync_copy(dst_h.at[pl.ds(eoff + ch * ECH, ECH)], sdst)

            def vec_body(v, _):
                s16 = ssrc[pl.ds(v * 16, 16)]
                d16 = sdst[pl.ds(v * 16, 16)]
                bv = lax.shift_right_logical(
                    (lax.shift_right_logical(d16, 10) * 21846), 16)
                dl = d16 - bv * BK
                valid = bv < NB
                rk, lastm = plsc.scan_count(bv, mask=valid)
                rank = rk - r0
                base = plsc.load_gather(cntv, [bv])
                pos = bv * CAPB + base + rank
                plsc.store_scatter(sflat, [pos], s16, mask=valid)
                plsc.store_scatter(dflat, [pos], dl, mask=valid)
                plsc.store_scatter(cntv, [bv], base + rank + 1,
                                   mask=lastm)
                return 0

            return lax.fori_loop(0, NVEC, vec_body, 0)

        lax.fori_loop(0, NECH, chunk_body, 0)

        for b in range(NB):
            pltpu.sync_copy(sflat.at[pl.ds(b * CAPB, CAPB)],
                            sl_h.at[wid, b, 0])
            pltpu.sync_copy(dflat.at[pl.ds(b * CAPB, CAPB)],
                            dl_h.at[wid, b, 0])
        pltpu.sync_copy(cntv, cn_h.at[wid, 0])

    return k(src1, dst1, zcap, tcap)


def _tc_launder(sl, dl, cn):
    def body(a_ref, b_ref, c_ref, ao_ref, bo_ref, co_ref):
        ao_ref[...] = a_ref[...]
        bo_ref[...] = b_ref[...]
        co_ref[...] = c_ref[...]

    sl2 = sl.reshape(NW * NB, CAPB)
    dl2 = dl.reshape(NW * NB, CAPB)
    cn2 = cn.reshape(NW, 64)
    outs = pl.pallas_call(
        body,
        out_shape=[
            jax.ShapeDtypeStruct((NW * NB, CAPB), jnp.int32),
            jax.ShapeDtypeStruct((NW * NB, CAPB), jnp.int32),
            jax.ShapeDtypeStruct((NW, 64), jnp.int32),
        ],
    )(sl2, dl2, cn2)
    return (outs[0].reshape(NW, NB, NBLK, 1, SEG),
            outs[1].reshape(NW, NB, NBLK, 1, SEG),
            outs[2].reshape(NW, 1, 64))


def _sc_scatter(x2, sl4, dl4, counts, zrows):
    mesh = plsc.VectorSubcoreMesh(core_axis_name="c", subcore_axis_name="s")

    @functools.partial(
        pl.kernel,
        out_type=[
            jax.ShapeDtypeStruct((NB, ACC, D), jnp.float32),
            jax.ShapeDtypeStruct((NB, NS, 1, ACC), jnp.float32),
        ],
        mesh=mesh,
        compiler_params=pltpu.CompilerParams(needs_layout_passes=False),
        scratch_types=[
            pltpu.VMEM((SEG,), jnp.int32),
            pltpu.VMEM((SEG,), jnp.int32),
            pltpu.VMEM((SEG, D), jnp.float32),
            pltpu.VMEM((ART, D), jnp.float32),
            pltpu.VMEM((ART, D), jnp.float32),
            pltpu.VMEM((NW, 1, 64), jnp.int32),
            pltpu.VMEM((ACC,), jnp.float32),
            pltpu.VMEM_SHARED((ACC, D), jnp.float32),
            pltpu.SemaphoreType.DMA,
        ],
    )
    def k(x_h, sl_h, dl_h, cn_h, z_h, agg_h, deg_h,
          sidx, didx, rows, zb, t2, cvm, degacc, acc_sh, sem):
        cid = lax.axis_index("c")
        sid = lax.axis_index("s")
        arow0 = sid * ART

        pltpu.sync_copy(cn_h, cvm)
        pltpu.sync_copy(z_h, zb)
        zeros16 = jnp.zeros((16,), jnp.float32)
        r0 = jnp.min(plsc.scan_count(jnp.zeros((16,), jnp.int32))[0])

        for p in range(NPASS):
            b = 2 * p + cid

            @pl.when(b < NB)
            def _():
                pltpu.sync_copy(zb, acc_sh.at[pl.ds(arow0, ART)])

                def dz(r, _):
                    degacc[pl.ds(r * 16, 16)] = zeros16
                    return 0
                lax.fori_loop(0, ACC // 16, dz, 0)

            plsc.subcore_barrier()

            @pl.when(b < NB)
            def _():
                for ti in range(2):
                    t = sid * 2 + ti
                    t16 = jnp.full((16,), t, jnp.int32)
                    z16 = jnp.zeros((16,), jnp.int32)
                    b16 = jnp.full((16,), b, jnp.int32)
                    cnt = jnp.max(plsc.load_gather(cvm, [t16, z16, b16]))

                    def blk(kk, _):
                        @pl.when(kk * SEG < cnt)
                        def _():
                            pltpu.sync_copy(sl_h.at[t, b, kk, 0], sidx)
                            pltpu.sync_copy(dl_h.at[t, b, kk, 0], didx)
                            pltpu.async_copy(
                                x_h.at[sidx], rows, sem).wait()
                            pltpu.sync_copy(
                                rows, acc_sh.at[didx], add=True)
                            for i in range(SEG // 16):
                                d16 = didx[pl.ds(i * 16, 16)]
                                rk, lastm = plsc.scan_count(d16)
                                cntf = (rk - r0 + 1).astype(jnp.float32)
                                plsc.addupdate_scatter(
                                    degacc, [d16], cntf, mask=lastm)
                        return 0
                    lax.fori_loop(0, NBLK, blk, 0)

            plsc.subcore_barrier()

            @pl.when(b < NB)
            def _():
                pltpu.sync_copy(acc_sh.at[pl.ds(arow0, ART)], t2)
                pltpu.sync_copy(t2, agg_h.at[b, pl.ds(arow0, ART)])
                pltpu.sync_copy(degacc, deg_h.at[b, sid, 0])

    return k(x2, sl4, dl4, counts, zrows)


def _fold_weights(Wm, Wu):
    def body(wm_ref, wu_ref, o_ref):
        wu2 = wu_ref[:, D:]
        o_ref[...] = lax.dot_general(
            wm_ref[...], wu2, (((0,), (1,)), ((), ())),
            preferred_element_type=jnp.float32)

    return pl.pallas_call(
        body,
        out_shape=jax.ShapeDtypeStruct((D, D), jnp.float32),
    )(Wm, Wu)


BN = 2000


def _update(x2d, agg, deg, Wu, Mt, bu, gamma, beta):
    def body(x_ref, a_ref, d_ref, wu_ref, mt_ref, bu_ref, g_ref, b_ref,
             o_ref):
        x = x_ref[...]
        degc = jnp.maximum(d_ref[...], 1.0)
        aggn = a_ref[...] / degc
        wu1 = wu_ref[:, :D]
        t = lax.dot_general(x, wu1, (((1,), (1,)), ((), ())),
                            preferred_element_type=jnp.float32)
        t = t + jnp.dot(aggn, mt_ref[...],
                        preferred_element_type=jnp.float32)
        t = t + bu_ref[...]
        h = 0.5 * t * (1.0 + lax.erf(t * (1.0 / math.sqrt(2.0)))) + x
        mean = jnp.mean(h, axis=-1, keepdims=True)
        c = h - mean
        var = jnp.mean(c * c, axis=-1, keepdims=True)
        o_ref[...] = c * lax.rsqrt(var + 1e-5) * g_ref[...] + b_ref[...]

    grid = (N // BN,)
    return pl.pallas_call(
        body,
        grid=grid,
        in_specs=[
            pl.BlockSpec((BN, D), lambda i: (i, 0)),
            pl.BlockSpec((BN, D), lambda i: (i, 0)),
            pl.BlockSpec((BN, 1), lambda i: (i, 0)),
            pl.BlockSpec((D, 2 * D), lambda i: (0, 0)),
            pl.BlockSpec((D, D), lambda i: (0, 0)),
            pl.BlockSpec((1, D), lambda i: (0, 0)),
            pl.BlockSpec((1, D), lambda i: (0, 0)),
            pl.BlockSpec((1, D), lambda i: (0, 0)),
        ],
        out_specs=pl.BlockSpec((BN, D), lambda i: (i, 0)),
        out_shape=jax.ShapeDtypeStruct((N, D), jnp.float32),
    )(x2d, agg, deg, Wu, Mt, bu, gamma, beta)


def kernel(x, edge_index, Wm, Wu, bu, gamma, beta):
    x2d = x.reshape(N, D)
    pad = EP - E
    src1 = jnp.concatenate(
        [edge_index[0], jnp.zeros((pad,), jnp.int32)])
    dst1 = jnp.concatenate(
        [edge_index[1], jnp.full((pad,), NB * BK, jnp.int32)])
    zcap = jnp.zeros((CAPB,), jnp.int32)
    tcap = jnp.full((CAPB,), TRASH, jnp.int32)
    zrows = jnp.zeros((ART, D), jnp.float32)

    sl, dl, counts = _sc_partition(src1, dst1, zcap, tcap)
    sl4, dl4, cn4 = _tc_launder(sl, dl, counts)

    aggB, degp = _sc_scatter(x2d, sl4, dl4, cn4, zrows)
    agg = aggB[:, :BK, :].reshape(NB * BK, D)[:N]
    deg = (degp.reshape(NB, NS, ACC).sum(axis=1)[:, :BK]
           .reshape(NB * BK)[:N].reshape(N, 1))

    Mt = _fold_weights(Wm, Wu)
    out2d = _update(x2d, agg, deg, Wu, Mt,
                    bu.reshape(1, D), gamma.reshape(1, D),
                    beta.reshape(1, D))
    return out2d.reshape(1, N, D)


# speedup vs baseline: 1.5346x; 1.0258x over previous
import functools
import math

import jax
import jax.numpy as jnp
from jax import lax
from jax.experimental import pallas as pl
from jax.experimental.pallas import tpu as pltpu
from jax.experimental.pallas import tpu_sc as plsc

N = 100000
D = 128
E = 600000
NS = 16
NC = 2
NW = NS * NC

EP = 614400
TEDGE = EP // NW
ECH = 960
NECH = TEDGE // ECH
NVEC = ECH // 16

BK = 3072
NB = 33
NBR = NB + 1
TRASH = BK
SEG = 112
NBLK = 9
CAPB = NBLK * SEG

ACC = 3200
ART = ACC // NS
NPASS = 17


def _sc_partition(src1, dst1, zcap, tcap):
    mesh = plsc.VectorSubcoreMesh(core_axis_name="c", subcore_axis_name="s")

    @functools.partial(
        pl.kernel,
        out_type=[
            jax.ShapeDtypeStruct((NW, NB, 1, CAPB), jnp.int32),
            jax.ShapeDtypeStruct((NW, NB, 1, CAPB), jnp.int32),
            jax.ShapeDtypeStruct((NW, 1, 64), jnp.int32),
        ],
        mesh=mesh,
        compiler_params=pltpu.CompilerParams(needs_layout_passes=False),
        scratch_types=[
            pltpu.VMEM((ECH,), jnp.int32),
            pltpu.VMEM((ECH,), jnp.int32),
            pltpu.VMEM((NBR * CAPB,), jnp.int32),
            pltpu.VMEM((NBR * CAPB,), jnp.int32),
            pltpu.VMEM((64,), jnp.int32),
        ],
    )
    def k(src_h, dst_h, z_h, t_h, sl_h, dl_h, cn_h,
          ssrc, sdst, sflat, dflat, cntv):
        cid = lax.axis_index("c")
        sid = lax.axis_index("s")
        wid = sid * NC + cid
        eoff = wid * TEDGE

        for b in range(NBR):
            pltpu.sync_copy(z_h, sflat.at[pl.ds(b * CAPB, CAPB)])
            pltpu.sync_copy(t_h, dflat.at[pl.ds(b * CAPB, CAPB)])
        zeros16 = jnp.zeros((16,), jnp.int32)
        for i in range(4):
            cntv[pl.ds(i * 16, 16)] = zeros16
        ones16 = jnp.ones((16,), jnp.int32)

        r0 = jnp.min(plsc.scan_count(zeros16)[0])

        def chunk_body(ch, _):
            pltpu.sync_copy(src_h.at[pl.ds(eoff + ch * ECH, ECH)], ssrc)
            pltpu.sync_copy(dst_h.at[pl.ds(eoff + ch * ECH, ECH)], sdst)

            def vec_body(v, _):
                s16 = ssrc[pl.ds(v * 16, 16)]
                d16 = sdst[pl.ds(v * 16, 16)]
                bv = lax.shift_right_logical(
                    (lax.shift_right_logical(d16, 10) * 21846), 16)
                dl = d16 - bv * BK
                valid = bv < NB
                rk, lastm = plsc.scan_count(bv, mask=valid)
                rank = rk - r0
                base = plsc.load_gather(cntv, [bv])
                pos = bv * CAPB + base + rank
                plsc.store_scatter(sflat, [pos], s16, mask=valid)
                plsc.store_scatter(dflat, [pos], dl, mask=valid)
                plsc.store_scatter(cntv, [bv], base + rank + 1,
                                   mask=lastm)
                return 0

            return lax.fori_loop(0, NVEC, vec_body, 0)

        lax.fori_loop(0, NECH, chunk_body, 0)

        for b in range(NB):
            pltpu.sync_copy(sflat.at[pl.ds(b * CAPB, CAPB)],
                            sl_h.at[wid, b, 0])
            pltpu.sync_copy(dflat.at[pl.ds(b * CAPB, CAPB)],
                            dl_h.at[wid, b, 0])
        pltpu.sync_copy(cntv, cn_h.at[wid, 0])

    return k(src1, dst1, zcap, tcap)


def _tc_launder(sl, dl, cn):
    def body(a_ref, b_ref, c_ref, ao_ref, bo_ref, co_ref):
        ao_ref[...] = a_ref[...]
        bo_ref[...] = b_ref[...]
        co_ref[...] = c_ref[...]

    sl2 = sl.reshape(NW * NB, CAPB)
    dl2 = dl.reshape(NW * NB, CAPB)
    cn2 = cn.reshape(NW, 64)
    outs = pl.pallas_call(
        body,
        out_shape=[
            jax.ShapeDtypeStruct((NW * NB, CAPB), jnp.int32),
            jax.ShapeDtypeStruct((NW * NB, CAPB), jnp.int32),
            jax.ShapeDtypeStruct((NW, 64), jnp.int32),
        ],
    )(sl2, dl2, cn2)
    return (outs[0].reshape(NW, NB, NBLK, SEG),
            outs[1].reshape(NW, NB, NBLK, SEG),
            outs[2].reshape(NW, 1, 64))


def _sc_scatter(x2, sl4, dl4, counts, zrows):
    mesh = plsc.VectorSubcoreMesh(core_axis_name="c", subcore_axis_name="s")

    @functools.partial(
        pl.kernel,
        out_type=[
            jax.ShapeDtypeStruct((NB, ACC, D), jnp.float32),
            jax.ShapeDtypeStruct((NB, NS, 1, ACC), jnp.float32),
        ],
        mesh=mesh,
        compiler_params=pltpu.CompilerParams(needs_layout_passes=False),
        scratch_types=[
            pltpu.VMEM((NBLK, SEG), jnp.int32),
            pltpu.VMEM((NBLK, SEG), jnp.int32),
            pltpu.VMEM((3, SEG, D), jnp.float32),
            pltpu.VMEM((ART, D), jnp.float32),
            pltpu.VMEM((NW, 1, 64), jnp.int32),
            pltpu.VMEM((ACC,), jnp.float32),
            pltpu.VMEM_SHARED((ACC, D), jnp.float32),
            pltpu.SemaphoreType.DMA,
            pltpu.SemaphoreType.DMA,
        ],
    )
    def k(x_h, sl_h, dl_h, cn_h, z_h, agg_h, deg_h,
          slist, dlist, rows, zb, cvm, degacc, acc_sh, semg, sems):
        cid = lax.axis_index("c")
        sid = lax.axis_index("s")
        arow0 = sid * ART

        pltpu.sync_copy(cn_h, cvm)
        pltpu.sync_copy(z_h, zb)
        zeros16 = jnp.zeros((16,), jnp.float32)
        r0 = jnp.min(plsc.scan_count(jnp.zeros((16,), jnp.int32))[0])

        def one_pass(p, _):
            b = 2 * p + cid

            @pl.when(b < NB)
            def _():
                pltpu.sync_copy(zb, acc_sh.at[pl.ds(arow0, ART)])

                def dz(r, _):
                    degacc[pl.ds(r * 16, 16)] = zeros16
                    return 0
                lax.fori_loop(0, ACC // 16, dz, 0)

            plsc.subcore_barrier()

            @pl.when(b < NB)
            def _():
                for ti in range(2):
                    t = sid * 2 + ti
                    t16 = jnp.full((16,), t, jnp.int32)
                    z16 = jnp.zeros((16,), jnp.int32)
                    b16 = jnp.full((16,), b, jnp.int32)
                    cnt = jnp.max(plsc.load_gather(cvm, [t16, z16, b16]))
                    pltpu.sync_copy(sl_h.at[t, b], slist)
                    pltpu.sync_copy(dl_h.at[t, b], dlist)

                    def gsrc(kk):
                        return x_h.at[slist.at[kk]]

                    def buf(kk):
                        return rows.at[kk % 3]

                    def sdst(kk):
                        return acc_sh.at[dlist.at[kk]]

                    @pl.when(0 < cnt)
                    def _():
                        pltpu.async_copy(gsrc(0), buf(0), semg)

                    for kk in range(NBLK):
                        @pl.when(kk * SEG < cnt)
                        def _(kk=kk):
                            pltpu.make_async_copy(
                                gsrc(kk), buf(kk), semg).wait()
                            if kk >= 2:
                                pltpu.make_async_copy(
                                    buf(kk - 2), sdst(kk - 2),
                                    sems).wait()
                            if kk + 1 < NBLK:
                                @pl.when((kk + 1) * SEG < cnt)
                                def _():
                                    pltpu.async_copy(
                                        gsrc(kk + 1), buf(kk + 1), semg)
                            pltpu.async_copy(buf(kk), sdst(kk), sems,
                                             add=True)
                            for i in range(SEG // 16):
                                d16 = dlist[kk, pl.ds(i * 16, 16)]
                                rk, lastm = plsc.scan_count(d16)
                                cntf = (rk - r0 + 1).astype(jnp.float32)
                                plsc.addupdate_scatter(
                                    degacc, [d16], cntf, mask=lastm)

                    for kk in range(NBLK):
                        tail = jnp.logical_and(kk * SEG < cnt,
                                               (kk + 2) * SEG >= cnt)

                        @pl.when(tail)
                        def _(kk=kk):
                            pltpu.make_async_copy(
                                buf(kk), sdst(kk), sems).wait()

            plsc.subcore_barrier()

            @pl.when(b < NB)
            def _():
                pltpu.sync_copy(acc_sh.at[pl.ds(arow0, ART)],
                                agg_h.at[b, pl.ds(arow0, ART)])
                pltpu.sync_copy(degacc, deg_h.at[b, sid, 0])
            return 0

        lax.fori_loop(0, NPASS, one_pass, 0)

    return k(x2, sl4, dl4, counts, zrows)


def _fold_weights(Wm, Wu):
    def body(wm_ref, wu_ref, o_ref):
        wu2 = wu_ref[:, D:]
        o_ref[...] = lax.dot_general(
            wm_ref[...], wu2, (((0,), (1,)), ((), ())),
            preferred_element_type=jnp.float32)

    return pl.pallas_call(
        body,
        out_shape=jax.ShapeDtypeStruct((D, D), jnp.float32),
    )(Wm, Wu)


BN = 2000


def _update(x2d, agg, deg, Wu, Mt, bu, gamma, beta):
    def body(x_ref, a_ref, d_ref, wu_ref, mt_ref, bu_ref, g_ref, b_ref,
             o_ref):
        x = x_ref[...]
        degc = jnp.maximum(d_ref[...], 1.0)
        aggn = a_ref[...] / degc
        wu1 = wu_ref[:, :D]
        t = lax.dot_general(x, wu1, (((1,), (1,)), ((), ())),
                            preferred_element_type=jnp.float32)
        t = t + jnp.dot(aggn, mt_ref[...],
                        preferred_element_type=jnp.float32)
        t = t + bu_ref[...]
        h = 0.5 * t * (1.0 + lax.erf(t * (1.0 / math.sqrt(2.0)))) + x
        mean = jnp.mean(h, axis=-1, keepdims=True)
        c = h - mean
        var = jnp.mean(c * c, axis=-1, keepdims=True)
        o_ref[...] = c * lax.rsqrt(var + 1e-5) * g_ref[...] + b_ref[...]

    grid = (N // BN,)
    return pl.pallas_call(
        body,
        grid=grid,
        in_specs=[
            pl.BlockSpec((BN, D), lambda i: (i, 0)),
            pl.BlockSpec((BN, D), lambda i: (i, 0)),
            pl.BlockSpec((BN, 1), lambda i: (i, 0)),
            pl.BlockSpec((D, 2 * D), lambda i: (0, 0)),
            pl.BlockSpec((D, D), lambda i: (0, 0)),
            pl.BlockSpec((1, D), lambda i: (0, 0)),
            pl.BlockSpec((1, D), lambda i: (0, 0)),
            pl.BlockSpec((1, D), lambda i: (0, 0)),
        ],
        out_specs=pl.BlockSpec((BN, D), lambda i: (i, 0)),
        out_shape=jax.ShapeDtypeStruct((N, D), jnp.float32),
    )(x2d, agg, deg, Wu, Mt, bu, gamma, beta)


def kernel(x, edge_index, Wm, Wu, bu, gamma, beta):
    x2d = x.reshape(N, D)
    pad = EP - E
    src1 = jnp.concatenate(
        [edge_index[0], jnp.zeros((pad,), jnp.int32)])
    dst1 = jnp.concatenate(
        [edge_index[1], jnp.full((pad,), NB * BK, jnp.int32)])
    zcap = jnp.zeros((CAPB,), jnp.int32)
    tcap = jnp.full((CAPB,), TRASH, jnp.int32)
    zrows = jnp.zeros((ART, D), jnp.float32)

    sl, dl, counts = _sc_partition(src1, dst1, zcap, tcap)
    sl4, dl4, cn4 = _tc_launder(sl, dl, counts)

    aggB, degp = _sc_scatter(x2d, sl4, dl4, cn4, zrows)
    agg = aggB[:, :BK, :].reshape(NB * BK, D)[:N]
    deg = (degp.reshape(NB, NS, ACC).sum(axis=1)[:, :BK]
           .reshape(NB * BK)[:N].reshape(N, 1))

    Mt = _fold_weights(Wm, Wu)
    out2d = _update(x2d, agg, deg, Wu, Mt,
                    bu.reshape(1, D), gamma.reshape(1, D),
                    beta.reshape(1, D))
    return out2d.reshape(1, N, D)


# speedup vs baseline: 1.5395x; 1.0032x over previous
import functools
import math

import jax
import jax.numpy as jnp
from jax import lax
from jax.experimental import pallas as pl
from jax.experimental.pallas import tpu as pltpu
from jax.experimental.pallas import tpu_sc as plsc

N = 100000
D = 128
E = 600000
NS = 16
NC = 2
NW = NS * NC

EP = 614400
TEDGE = EP // NW
ECH = 960
NECH = TEDGE // ECH
NVEC = ECH // 16

BK = 3072
NB = 33
NBR = NB + 1
TRASH = BK
SEG = 112
NBLK = 9
CAPB = NBLK * SEG

ACC = 3200
ART = ACC // NS
NPASS = 17


def _sc_partition(src1, dst1, zcap, tcap):
    mesh = plsc.VectorSubcoreMesh(core_axis_name="c", subcore_axis_name="s")

    @functools.partial(
        pl.kernel,
        out_type=[
            jax.ShapeDtypeStruct((NW, NB, 1, CAPB), jnp.int32),
            jax.ShapeDtypeStruct((NW, NB, 1, CAPB), jnp.int32),
            jax.ShapeDtypeStruct((NW, 1, 64), jnp.int32),
        ],
        mesh=mesh,
        compiler_params=pltpu.CompilerParams(needs_layout_passes=False),
        scratch_types=[
            pltpu.VMEM((ECH,), jnp.int32),
            pltpu.VMEM((ECH,), jnp.int32),
            pltpu.VMEM((NBR * CAPB,), jnp.int32),
            pltpu.VMEM((NBR * CAPB,), jnp.int32),
            pltpu.VMEM((64,), jnp.int32),
        ],
    )
    def k(src_h, dst_h, z_h, t_h, sl_h, dl_h, cn_h,
          ssrc, sdst, sflat, dflat, cntv):
        cid = lax.axis_index("c")
        sid = lax.axis_index("s")
        wid = sid * NC + cid
        eoff = wid * TEDGE

        for b in range(NBR):
            pltpu.sync_copy(z_h, sflat.at[pl.ds(b * CAPB, CAPB)])
            pltpu.sync_copy(t_h, dflat.at[pl.ds(b * CAPB, CAPB)])
        zeros16 = jnp.zeros((16,), jnp.int32)
        for i in range(4):
            cntv[pl.ds(i * 16, 16)] = zeros16
        ones16 = jnp.ones((16,), jnp.int32)

        r0 = jnp.min(plsc.scan_count(zeros16)[0])

        def chunk_body(ch, _):
            pltpu.sync_copy(src_h.at[pl.ds(eoff + ch * ECH, ECH)], ssrc)
            pltpu.sync_copy(dst_h.at[pl.ds(eoff + ch * ECH, ECH)], sdst)

            def vec_body(v, _):
                s16 = ssrc[pl.ds(v * 16, 16)]
                d16 = sdst[pl.ds(v * 16, 16)]
                bv = lax.shift_right_logical(
                    (lax.shift_right_logical(d16, 10) * 21846), 16)
                dl = d16 - bv * BK
                valid = bv < NB
                rk, lastm = plsc.scan_count(bv, mask=valid)
                rank = rk - r0
                base = plsc.load_gather(cntv, [bv])
                pos = bv * CAPB + base + rank
                plsc.store_scatter(sflat, [pos], s16, mask=valid)
                plsc.store_scatter(dflat, [pos], dl, mask=valid)
                plsc.store_scatter(cntv, [bv], base + rank + 1,
                                   mask=lastm)
                return 0

            return lax.fori_loop(0, NVEC, vec_body, 0)

        lax.fori_loop(0, NECH, chunk_body, 0)

        for b in range(NB):
            pltpu.sync_copy(sflat.at[pl.ds(b * CAPB, CAPB)],
                            sl_h.at[wid, b, 0])
            pltpu.sync_copy(dflat.at[pl.ds(b * CAPB, CAPB)],
                            dl_h.at[wid, b, 0])
        pltpu.sync_copy(cntv, cn_h.at[wid, 0])

    return k(src1, dst1, zcap, tcap)


def _tc_launder(sl, dl, cn):
    def body(a_ref, b_ref, c_ref, ao_ref, bo_ref, co_ref):
        ao_ref[...] = a_ref[...]
        bo_ref[...] = b_ref[...]
        co_ref[...] = c_ref[...]

    sl2 = sl.reshape(NW * NB, CAPB)
    dl2 = dl.reshape(NW * NB, CAPB)
    cn2 = cn.reshape(NW, 64)
    outs = pl.pallas_call(
        body,
        out_shape=[
            jax.ShapeDtypeStruct((NW * NB, CAPB), jnp.int32),
            jax.ShapeDtypeStruct((NW * NB, CAPB), jnp.int32),
            jax.ShapeDtypeStruct((NW, 64), jnp.int32),
        ],
    )(sl2, dl2, cn2)
    return (outs[0].reshape(NW, NB, NBLK, SEG),
            outs[1].reshape(NW, NB, NBLK, SEG),
            outs[2].reshape(NW, 1, 64))


def _sc_scatter(x2, sl4, dl4, counts, zrows):
    mesh = plsc.VectorSubcoreMesh(core_axis_name="c", subcore_axis_name="s")

    @functools.partial(
        pl.kernel,
        out_type=[
            jax.ShapeDtypeStruct((NB, ACC, D), jnp.float32),
            jax.ShapeDtypeStruct((NB, NS, 1, ACC), jnp.float32),
        ],
        mesh=mesh,
        compiler_params=pltpu.CompilerParams(needs_layout_passes=False),
        scratch_types=[
            pltpu.VMEM((NBLK, SEG), jnp.int32),
            pltpu.VMEM((NBLK, SEG), jnp.int32),
            pltpu.VMEM((4, SEG, D), jnp.float32),
            pltpu.VMEM((NW, 1, 64), jnp.int32),
            pltpu.VMEM((ACC,), jnp.float32),
            pltpu.VMEM_SHARED((ACC, D), jnp.float32),
            pltpu.SemaphoreType.DMA,
            pltpu.SemaphoreType.DMA,
        ],
    )
    def k(x_h, sl_h, dl_h, cn_h, z_h, agg_h, deg_h,
          slist, dlist, rows, cvm, degacc, acc_sh, semg, sems):
        cid = lax.axis_index("c")
        sid = lax.axis_index("s")
        arow0 = sid * ART

        pltpu.sync_copy(cn_h, cvm)
        zeros16 = jnp.zeros((16,), jnp.float32)
        r0 = jnp.min(plsc.scan_count(jnp.zeros((16,), jnp.int32))[0])

        def one_pass(p, _):
            b = 2 * p + cid

            @pl.when(b < NB)
            def _():
                pltpu.sync_copy(z_h, acc_sh.at[pl.ds(arow0, ART)])

                def dz(r, _):
                    degacc[pl.ds(r * 16, 16)] = zeros16
                    return 0
                lax.fori_loop(0, ACC // 16, dz, 0)

            plsc.subcore_barrier()

            @pl.when(b < NB)
            def _():
                for ti in range(2):
                    t = sid * 2 + ti
                    t16 = jnp.full((16,), t, jnp.int32)
                    z16 = jnp.zeros((16,), jnp.int32)
                    b16 = jnp.full((16,), b, jnp.int32)
                    cnt = jnp.max(plsc.load_gather(cvm, [t16, z16, b16]))
                    pltpu.sync_copy(sl_h.at[t, b], slist)
                    pltpu.sync_copy(dl_h.at[t, b], dlist)

                    def gsrc(kk):
                        return x_h.at[slist.at[kk]]

                    def buf(kk):
                        return rows.at[kk % 4]

                    def sdst(kk):
                        return acc_sh.at[dlist.at[kk]]

                    @pl.when(0 < cnt)
                    def _():
                        pltpu.async_copy(gsrc(0), buf(0), semg)

                    @pl.when(SEG < cnt)
                    def _():
                        pltpu.async_copy(gsrc(1), buf(1), semg)

                    for kk in range(NBLK):
                        @pl.when(kk * SEG < cnt)
                        def _(kk=kk):
                            pltpu.make_async_copy(
                                gsrc(kk), buf(kk), semg).wait()
                            if kk >= 2:
                                pltpu.make_async_copy(
                                    buf(kk - 2), sdst(kk - 2),
                                    sems).wait()
                            if kk + 2 < NBLK:
                                @pl.when((kk + 2) * SEG < cnt)
                                def _():
                                    pltpu.async_copy(
                                        gsrc(kk + 2), buf(kk + 2), semg)
                            pltpu.async_copy(buf(kk), sdst(kk), sems,
                                             add=True)
                            for i in range(SEG // 16):
                                d16 = dlist[kk, pl.ds(i * 16, 16)]
                                rk, lastm = plsc.scan_count(d16)
                                cntf = (rk - r0 + 1).astype(jnp.float32)
                                plsc.addupdate_scatter(
                                    degacc, [d16], cntf, mask=lastm)

                    for kk in range(NBLK):
                        tail = jnp.logical_and(kk * SEG < cnt,
                                               (kk + 2) * SEG >= cnt)

                        @pl.when(tail)
                        def _(kk=kk):
                            pltpu.make_async_copy(
                                buf(kk), sdst(kk), sems).wait()

            plsc.subcore_barrier()

            @pl.when(b < NB)
            def _():
                pltpu.sync_copy(acc_sh.at[pl.ds(arow0, ART)],
                                agg_h.at[b, pl.ds(arow0, ART)])
                pltpu.sync_copy(degacc, deg_h.at[b, sid, 0])
            return 0

        lax.fori_loop(0, NPASS, one_pass, 0)

    return k(x2, sl4, dl4, counts, zrows)


def _fold_weights(Wm, Wu):
    def body(wm_ref, wu_ref, o_ref):
        wu2 = wu_ref[:, D:]
        o_ref[...] = lax.dot_general(
            wm_ref[...], wu2, (((0,), (1,)), ((), ())),
            preferred_element_type=jnp.float32)

    return pl.pallas_call(
        body,
        out_shape=jax.ShapeDtypeStruct((D, D), jnp.float32),
    )(Wm, Wu)


BN = 2000


def _update(x2d, agg, deg, Wu, Mt, bu, gamma, beta):
    def body(x_ref, a_ref, d_ref, wu_ref, mt_ref, bu_ref, g_ref, b_ref,
             o_ref):
        x = x_ref[...]
        degc = jnp.maximum(d_ref[...], 1.0)
        aggn = a_ref[...] / degc
        wu1 = wu_ref[:, :D]
        t = lax.dot_general(x, wu1, (((1,), (1,)), ((), ())),
                            preferred_element_type=jnp.float32)
        t = t + jnp.dot(aggn, mt_ref[...],
                        preferred_element_type=jnp.float32)
        t = t + bu_ref[...]
        h = 0.5 * t * (1.0 + lax.erf(t * (1.0 / math.sqrt(2.0)))) + x
        mean = jnp.mean(h, axis=-1, keepdims=True)
        c = h - mean
        var = jnp.mean(c * c, axis=-1, keepdims=True)
        o_ref[...] = c * lax.rsqrt(var + 1e-5) * g_ref[...] + b_ref[...]

    grid = (N // BN,)
    return pl.pallas_call(
        body,
        grid=grid,
        in_specs=[
            pl.BlockSpec((BN, D), lambda i: (i, 0)),
            pl.BlockSpec((BN, D), lambda i: (i, 0)),
            pl.BlockSpec((BN, 1), lambda i: (i, 0)),
            pl.BlockSpec((D, 2 * D), lambda i: (0, 0)),
            pl.BlockSpec((D, D), lambda i: (0, 0)),
            pl.BlockSpec((1, D), lambda i: (0, 0)),
            pl.BlockSpec((1, D), lambda i: (0, 0)),
            pl.BlockSpec((1, D), lambda i: (0, 0)),
        ],
        out_specs=pl.BlockSpec((BN, D), lambda i: (i, 0)),
        out_shape=jax.ShapeDtypeStruct((N, D), jnp.float32),
    )(x2d, agg, deg, Wu, Mt, bu, gamma, beta)


def kernel(x, edge_index, Wm, Wu, bu, gamma, beta):
    x2d = x.reshape(N, D)
    pad = EP - E
    src1 = jnp.concatenate(
        [edge_index[0], jnp.zeros((pad,), jnp.int32)])
    dst1 = jnp.concatenate(
        [edge_index[1], jnp.full((pad,), NB * BK, jnp.int32)])
    zcap = jnp.zeros((CAPB,), jnp.int32)
    tcap = jnp.full((CAPB,), TRASH, jnp.int32)
    zrows = jnp.zeros((ART, D), jnp.float32)

    sl, dl, counts = _sc_partition(src1, dst1, zcap, tcap)
    sl4, dl4, cn4 = _tc_launder(sl, dl, counts)

    aggB, degp = _sc_scatter(x2d, sl4, dl4, cn4, zrows)
    agg = aggB[:, :BK, :].reshape(NB * BK, D)[:N]
    deg = (degp.reshape(NB, NS, ACC).sum(axis=1)[:, :BK]
           .reshape(NB * BK)[:N].reshape(N, 1))

    Mt = _fold_weights(Wm, Wu)
    out2d = _update(x2d, agg, deg, Wu, Mt,
                    bu.reshape(1, D), gamma.reshape(1, D),
                    beta.reshape(1, D))
    return out2d.reshape(1, N, D)
